# baseline (speedup 1.0000x reference)
import functools

import jax
import jax.numpy as jnp
from jax import lax
from jax.experimental import pallas as pl
from jax.experimental.pallas import tpu as pltpu
from jax.experimental.pallas import tpu_sc as plsc

VOCAB = 1000000
EMBD = 64
B = 16384

NC = 2
NS = 16
L = 16
NW = NC * NS
BPW = B // NW
CHUNK = 128
NCH = BPW // CHUNK
GROUPS = BPW // L

_mesh = plsc.VectorSubcoreMesh(core_axis_name="c", subcore_axis_name="s")


@functools.partial(
    pl.kernel,
    out_type=jax.ShapeDtypeStruct((NW, BPW), jnp.float32),
    mesh=_mesh,
    compiler_params=pltpu.CompilerParams(
        needs_layout_passes=False, use_tc_tiling_on_sc=False),
    scratch_types=[
        pltpu.VMEM((NCH, CHUNK), jnp.int32),
        pltpu.VMEM((NCH, CHUNK), jnp.int32),
        pltpu.VMEM((BPW, EMBD), jnp.float32),
        pltpu.VMEM((BPW, EMBD), jnp.float32),
        pltpu.VMEM((L * L,), jnp.float32),
        pltpu.VMEM((BPW,), jnp.float32),
        pltpu.SemaphoreType.DMA,
    ],
)
def _skipgram_sc(focus_hbm, context_hbm, emb_hbm, out_hbm,
                 fidx, cidx, frows, crows, part, outv, sem):
    wid = lax.axis_index("s") * NC + lax.axis_index("c")

    pltpu.sync_copy(focus_hbm.at[wid], fidx)
    pltpu.sync_copy(context_hbm.at[wid], cidx)

    copies = []
    for j in range(NCH):
        dst = frows.at[pl.ds(j * CHUNK, CHUNK)]
        copies.append(pltpu.async_copy(emb_hbm.at[fidx.at[j]], dst, sem))
    for j in range(NCH):
        dst = crows.at[pl.ds(j * CHUNK, CHUNK)]
        copies.append(pltpu.async_copy(emb_hbm.at[cidx.at[j]], dst, sem))
    for c in copies:
        c.wait()

    iota = lax.iota(jnp.int32, L)
    col_base = iota * L

    def body(g, _):
        base = g * L
        for rr in range(L):
            fr = frows.at[base + rr]
            cr = crows.at[base + rr]
            acc = fr[pl.ds(0, L)] * cr[pl.ds(0, L)]
            for k in range(1, EMBD // L):
                acc = acc + fr[pl.ds(k * L, L)] * cr[pl.ds(k * L, L)]
            plsc.store_scatter(part, [col_base + rr], acc)
        tot = part[pl.ds(0, L)]
        for c in range(1, L):
            tot = tot + part[pl.ds(c * L, L)]
        outv[pl.ds(base, L)] = tot
        return _

    lax.fori_loop(0, GROUPS, body, None)

    pltpu.sync_copy(outv, out_hbm.at[wid])


def kernel(focus, context, embeddings):
    focus = focus.reshape(NW, NCH, CHUNK)
    context = context.reshape(NW, NCH, CHUNK)
    out = _skipgram_sc(focus, context, embeddings)
    return out.reshape(B)
